# Initial kernel scaffold; baseline (speedup 1.0000x reference)
import jax
import jax.numpy as jnp
from jax.experimental import pallas as pl


def kernel(pred, gt):
    raise NotImplementedError("write your pallas kernel here")


# speedup vs baseline: 1.3113x; 1.3113x over previous
import functools

import jax
import jax.numpy as jnp
from jax import lax
from jax.experimental import pallas as pl
from jax.experimental.pallas import tpu as pltpu
from jax.experimental.pallas import tpu_sc as plsc

NB = 1024
NCOPY = 16
BLK = 2048
NTILES = 32


def _loss_body(pred_ref, gt_ref, cls_ref, bb_ref, lm_ref):
    p = pred_ref[...]
    g = gt_ref[...]
    col = lax.broadcasted_iota(jnp.int32, (1, 15), 1)
    lab = jnp.sum(jnp.where(col == 0, g, 0.0), axis=1, keepdims=True)
    x = jnp.sum(jnp.where(col == 0, p, 0.0), axis=1, keepdims=True)
    y = jnp.where(lab == 1.0, 1.0, 0.0)
    bce = jnp.maximum(x, 0.0) - x * y + jnp.log1p(jnp.exp(-jnp.abs(x)))
    d = p - g
    sq = d * d
    lb = jnp.sum(jnp.where((col >= 1) & (col <= 4), sq, 0.0),
                 axis=1, keepdims=True) * 0.25
    ll = jnp.sum(jnp.where(col >= 5, sq, 0.0), axis=1, keepdims=True) * 0.1
    mask_b = jnp.logical_or(lab == 1.0, lab == -1.0)
    mask_l = lab == -2.0
    cls_ref[...] = bce
    bb_ref[...] = jnp.where(mask_b, lb, -1.0)
    lm_ref[...] = jnp.where(mask_l, ll, -1.0)


def _losses(pred, gt):
    n = pred.shape[0]
    out = jax.ShapeDtypeStruct((n, 1), jnp.float32)
    return pl.pallas_call(
        _loss_body,
        grid=(n // BLK,),
        in_specs=[
            pl.BlockSpec((BLK, 15), lambda i: (i, 0)),
            pl.BlockSpec((BLK, 15), lambda i: (i, 0)),
        ],
        out_specs=[pl.BlockSpec((BLK, 1), lambda i: (i, 0))] * 3,
        out_shape=[out, out, out],
    )(pred, gt)


def _mono_key(v):
    bits = lax.bitcast_convert_type(v, jnp.int32)
    return jnp.where(bits < 0, bits ^ jnp.int32(0x7FFFFFFF), bits)


def _hist_core(stage, hc, hs, mc, ms, n_vec, bucket_fn):
    z16i = jnp.zeros((16,), jnp.int32)
    z16f = jnp.zeros((16,), jnp.float32)
    ones = jnp.ones((16,), jnp.int32)
    lane_off = lax.iota(jnp.int32, 16) * NB

    def zero_body(i, _):
        hc[pl.ds(i * 16, 16)] = z16i
        hs[pl.ds(i * 16, 16)] = z16f
        return 0
    lax.fori_loop(0, NB * NCOPY // 16, zero_body, 0)

    def scat_body(i, _):
        v = stage[pl.ds(i * 16, 16)]
        b, mask = bucket_fn(v)
        idx = lane_off + b
        plsc.addupdate_scatter(hc, [idx], ones, mask=mask)
        plsc.addupdate_scatter(hs, [idx], v, mask=mask)
        return 0
    lax.fori_loop(0, n_vec, scat_body, 0)

    def merge_body(i, _):
        acc_c = hc[pl.ds(i * 16, 16)]
        acc_s = hs[pl.ds(i * 16, 16)]
        for l in range(1, NCOPY):
            acc_c = acc_c + hc[pl.ds(l * NB + i * 16, 16)]
            acc_s = acc_s + hs[pl.ds(l * NB + i * 16, 16)]
        mc[pl.ds(i * 16, 16)] = acc_c
        ms[pl.ds(i * 16, 16)] = acc_s
        return 0
    lax.fori_loop(0, NB // 16, merge_body, 0)


def _make_hist1(n):
    chunk = n // NTILES
    mesh = plsc.VectorSubcoreMesh(core_axis_name="c", subcore_axis_name="s")

    @functools.partial(
        pl.kernel,
        mesh=mesh,
        compiler_params=pltpu.CompilerParams(needs_layout_passes=False),
        out_type=[
            jax.ShapeDtypeStruct((NTILES * 3 * NB,), jnp.int32),
            jax.ShapeDtypeStruct((NTILES * 3 * NB,), jnp.float32),
        ],
        scratch_types=[
            pltpu.VMEM((chunk,), jnp.float32),
            pltpu.VMEM((NB * NCOPY,), jnp.int32),
            pltpu.VMEM((NB * NCOPY,), jnp.float32),
            pltpu.VMEM((NB,), jnp.int32),
            pltpu.VMEM((NB,), jnp.float32),
        ],
    )
    def hist1(l0, l1, l2, cnt_out, sum_out, stage, hc, hs, mc, ms):
        t = lax.axis_index("c") * 16 + lax.axis_index("s")
        base = t * chunk

        def bucket_fn(v):
            key = _mono_key(v)
            return (key >> 22) + 512, None

        for a, ref in enumerate((l0, l1, l2)):
            pltpu.sync_copy(ref.at[pl.ds(base, chunk)], stage)
            _hist_core(stage, hc, hs, mc, ms, chunk // 16, bucket_fn)
            pltpu.sync_copy(mc, cnt_out.at[pl.ds((t * 3 + a) * NB, NB)])
            pltpu.sync_copy(ms, sum_out.at[pl.ds((t * 3 + a) * NB, NB)])

    return hist1


def _make_hist2(n):
    chunk = n // NTILES
    mesh = plsc.VectorSubcoreMesh(core_axis_name="c", subcore_axis_name="s")

    @functools.partial(
        pl.kernel,
        mesh=mesh,
        compiler_params=pltpu.CompilerParams(needs_layout_passes=False),
        out_type=[
            jax.ShapeDtypeStruct((NTILES * 3 * NB,), jnp.int32),
            jax.ShapeDtypeStruct((NTILES * 3 * NB,), jnp.float32),
        ],
        scratch_types=[
            pltpu.VMEM((chunk,), jnp.float32),
            pltpu.VMEM((8, 128), jnp.float32),
            pltpu.VMEM((NB * NCOPY,), jnp.int32),
            pltpu.VMEM((NB * NCOPY,), jnp.float32),
            pltpu.VMEM((NB,), jnp.int32),
            pltpu.VMEM((NB,), jnp.float32),
        ],
    )
    def hist2(l0, l1, l2, st, cnt_out, sum_out, stage, stv, hc, hs, mc, ms):
        t = lax.axis_index("c") * 16 + lax.axis_index("s")
        base = t * chunk
        pltpu.sync_copy(st, stv)

        for a, ref in enumerate((l0, l1, l2)):
            b1 = stv[a, pl.ds(0, 16)][0].astype(jnp.int32)

            def bucket_fn(v, b1=b1):
                key = _mono_key(v)
                match = ((key >> 22) + 512) == b1
                return (key >> 12) & (NB - 1), match

            pltpu.sync_copy(ref.at[pl.ds(base, chunk)], stage)
            _hist_core(stage, hc, hs, mc, ms, chunk // 16, bucket_fn)
            pltpu.sync_copy(mc, cnt_out.at[pl.ds((t * 3 + a) * NB, NB)])
            pltpu.sync_copy(ms, sum_out.at[pl.ds((t * 3 + a) * NB, NB)])

    return hist2


def _suffix(x, tri):
    return lax.dot_general(x, tri, (((1,), (1,)), ((), ())),
                           preferred_element_type=jnp.float32)


def _tri():
    ii = lax.broadcasted_iota(jnp.int32, (NB, NB), 0)
    jj = lax.broadcasted_iota(jnp.int32, (NB, NB), 1)
    return (jj >= ii).astype(jnp.float32)


def _scan1_body(k0, cnt_ref, sum_ref, st_ref):
    cnt = jnp.sum(cnt_ref[...].astype(jnp.float32), axis=0)
    ssum = jnp.sum(sum_ref[...], axis=0)
    tri = _tri()
    jr = lax.broadcasted_iota(jnp.int32, (1, NB), 1)
    rows = []
    for a in range(3):
        ca = cnt[a:a + 1, :]
        sa = ssum[a:a + 1, :]
        rc = _suffix(ca, tri)
        if a == 0:
            k = jnp.float32(k0)
        else:
            cv = jnp.sum(jnp.where(jr >= 512, ca, 0.0))
            k = jnp.floor(cv * 7.0 / 10.0)
        b1 = jnp.sum((rc >= k).astype(jnp.float32)) - 1.0
        gt = jr.astype(jnp.float32) > b1
        count_above = jnp.sum(jnp.where(gt, ca, 0.0))
        sum_above = jnp.sum(jnp.where(gt, sa, 0.0))
        rows.append((b1, count_above, sum_above, k))
    r8 = lax.broadcasted_iota(jnp.int32, (8, 128), 0)
    c128 = lax.broadcasted_iota(jnp.int32, (8, 128), 1)
    st = jnp.zeros((8, 128), jnp.float32)
    for a in range(3):
        for j, val in enumerate(rows[a]):
            st = jnp.where((r8 == a) & (c128 == j), val, st)
    st_ref[...] = st


def _scan1(cnt1, sum1, k0):
    return pl.pallas_call(
        functools.partial(_scan1_body, k0),
        out_shape=jax.ShapeDtypeStruct((8, 128), jnp.float32),
    )(cnt1, sum1)


def _scan2_body(cnt_ref, sum_ref, st_ref, out_ref):
    cnt = jnp.sum(cnt_ref[...].astype(jnp.float32), axis=0)
    ssum = jnp.sum(sum_ref[...], axis=0)
    st = st_ref[...]
    tri = _tri()
    jr = lax.broadcasted_iota(jnp.int32, (1, NB), 1)
    r8 = lax.broadcasted_iota(jnp.int32, (8, 128), 0)
    c128 = lax.broadcasted_iota(jnp.int32, (8, 128), 1)
    losses = []
    for a in range(3):
        def pick(j):
            return jnp.sum(jnp.where((r8 == a) & (c128 == j), st, 0.0))
        count_above, sum_above, k = pick(1), pick(2), pick(3)
        ca = cnt[a:a + 1, :]
        sa = ssum[a:a + 1, :]
        rc = _suffix(ca, tri)
        b2 = jnp.sum((count_above + rc >= k).astype(jnp.float32)) - 1.0
        jf = jr.astype(jnp.float32)
        count_gt = count_above + jnp.sum(jnp.where(jf > b2, ca, 0.0))
        sum_gt = sum_above + jnp.sum(jnp.where(jf > b2, sa, 0.0))
        count_eq = jnp.sum(jnp.where(jf == b2, ca, 0.0))
        sum_eq = jnp.sum(jnp.where(jf == b2, sa, 0.0))
        r = k - count_gt
        losses.append((sum_gt + r * (sum_eq / count_eq)) / k)
    total = losses[0] + 0.5 * losses[1] + 0.5 * losses[2]
    vals = (total, losses[0], losses[1], losses[2])
    out = jnp.zeros((8, 128), jnp.float32)
    for j, v in enumerate(vals):
        out = jnp.where((r8 == 0) & (c128 == j), v, out)
    out_ref[...] = out


def _scan2(cnt2, sum2, st):
    return pl.pallas_call(
        _scan2_body,
        out_shape=jax.ShapeDtypeStruct((8, 128), jnp.float32),
    )(cnt2, sum2, st)


def kernel(pred, gt):
    n = pred.shape[0]
    k0 = int(0.7 * n)
    l_cls, l_bb, l_lm = _losses(pred, gt)
    l_cls = l_cls.reshape(n)
    l_bb = l_bb.reshape(n)
    l_lm = l_lm.reshape(n)
    cnt1, sum1 = _make_hist1(n)(l_cls, l_bb, l_lm)
    cnt1 = cnt1.reshape(NTILES, 3, NB)
    sum1 = sum1.reshape(NTILES, 3, NB)
    st = _scan1(cnt1, sum1, k0)
    cnt2, sum2 = _make_hist2(n)(l_cls, l_bb, l_lm, st)
    cnt2 = cnt2.reshape(NTILES, 3, NB)
    sum2 = sum2.reshape(NTILES, 3, NB)
    out = _scan2(cnt2, sum2, st)
    return out[0, 0], out[0, 1], out[0, 2], out[0, 3]


# speedup vs baseline: 2.5541x; 1.9477x over previous
import functools

import jax
import jax.numpy as jnp
from jax import lax
from jax.experimental import pallas as pl
from jax.experimental.pallas import tpu as pltpu
from jax.experimental.pallas import tpu_sc as plsc

NB = 1024
NCOPY = 16
BLK = 4096
NTILES = 32
LANES = 128


BR = 32


def _loss_body(pred_ref, gt_ref, e_ref, wb_ref, wl_ref, cls_ref, bb_ref,
               lm_ref):
    p = pred_ref[...]
    g = gt_ref[...]
    d = p - g
    sq = d * d
    dot = lambda a, b: lax.dot_general(a, b, (((1,), (0,)), ((), ())),
                                       preferred_element_type=jnp.float32)
    x = dot(p, e_ref[...])
    lab = dot(g, e_ref[...])
    lb = dot(sq, wb_ref[...])
    ll = dot(sq, wl_ref[...])
    y = jnp.where(lab == 1.0, 1.0, 0.0)
    bce = jnp.maximum(x, 0.0) - x * y + jnp.log1p(jnp.exp(-jnp.abs(x)))
    cls_ref[...] = bce
    bb_ref[...] = jnp.where(jnp.logical_or(lab == 1.0, lab == -1.0), lb, -1.0)
    lm_ref[...] = jnp.where(lab == -2.0, ll, -1.0)


def _losses(pred, gt):
    n = pred.shape[0]
    nr = n // LANES
    pc = 15 * LANES
    p2 = pred.reshape(nr, pc)
    g2 = gt.reshape(nr, pc)
    l = jnp.arange(pc)
    rr = l // 15
    cc = l % 15
    onehot = (rr[:, None] == jnp.arange(LANES)[None, :]).astype(jnp.float32)
    e = onehot * (cc == 0)[:, None]
    wb = onehot * (((cc >= 1) & (cc <= 4)) * 0.25)[:, None]
    wl = onehot * ((cc >= 5) * 0.1)[:, None]
    out = jax.ShapeDtypeStruct((nr, LANES), jnp.float32)
    cidx = lambda i: (0, 0)
    return pl.pallas_call(
        _loss_body,
        grid=(nr // BR,),
        in_specs=[
            pl.BlockSpec((BR, pc), lambda i: (i, 0)),
            pl.BlockSpec((BR, pc), lambda i: (i, 0)),
            pl.BlockSpec((pc, LANES), cidx),
            pl.BlockSpec((pc, LANES), cidx),
            pl.BlockSpec((pc, LANES), cidx),
        ],
        out_specs=[pl.BlockSpec((BR, LANES), lambda i: (i, 0))] * 3,
        out_shape=[out, out, out],
    )(p2, g2, e.astype(jnp.float32), wb.astype(jnp.float32),
      wl.astype(jnp.float32))


def _mono_key(v):
    bits = lax.bitcast_convert_type(v, jnp.int32)
    return jnp.where(bits < 0, bits ^ jnp.int32(0x7FFFFFFF), bits)


def _hist_core(stage, hc, hs, mc, ms, n_vec, bucket_fn):
    z16i = jnp.zeros((16,), jnp.int32)
    z16f = jnp.zeros((16,), jnp.float32)
    ones = jnp.ones((16,), jnp.int32)
    lane_off = lax.iota(jnp.int32, 16) * NB

    def zero_body(i, _):
        hc[pl.ds(i * 16, 16)] = z16i
        hs[pl.ds(i * 16, 16)] = z16f
        return 0
    lax.fori_loop(0, NB * NCOPY // 16, zero_body, 0)

    def scat_body(i, _):
        v = stage[i >> 3, pl.ds((i & 7) * 16, 16)]
        b, mask = bucket_fn(v)
        idx = lane_off + b
        plsc.addupdate_scatter(hc, [idx], ones, mask=mask)
        plsc.addupdate_scatter(hs, [idx], v, mask=mask)
        return 0
    lax.fori_loop(0, n_vec, scat_body, 0)

    def merge_body(i, _):
        acc_c = hc[pl.ds(i * 16, 16)]
        acc_s = hs[pl.ds(i * 16, 16)]
        for l in range(1, NCOPY):
            acc_c = acc_c + hc[pl.ds(l * NB + i * 16, 16)]
            acc_s = acc_s + hs[pl.ds(l * NB + i * 16, 16)]
        mc[0, pl.ds(i * 16, 16)] = acc_c
        ms[0, pl.ds(i * 16, 16)] = acc_s
        return 0
    lax.fori_loop(0, NB // 16, merge_body, 0)


def _make_hist1(n):
    rows = n // LANES // NTILES
    mesh = plsc.VectorSubcoreMesh(core_axis_name="c", subcore_axis_name="s")

    @functools.partial(
        pl.kernel,
        mesh=mesh,
        compiler_params=pltpu.CompilerParams(needs_layout_passes=False),
        out_type=[
            jax.ShapeDtypeStruct((NTILES, 3 * NB), jnp.int32),
            jax.ShapeDtypeStruct((NTILES, 3 * NB), jnp.float32),
        ],
        scratch_types=[
            pltpu.VMEM((rows, LANES), jnp.float32),
            pltpu.VMEM((NB * NCOPY,), jnp.int32),
            pltpu.VMEM((NB * NCOPY,), jnp.float32),
            pltpu.VMEM((1, NB), jnp.int32),
            pltpu.VMEM((1, NB), jnp.float32),
        ],
    )
    def hist1(l0, l1, l2, cnt_out, sum_out, stage, hc, hs, mc, ms):
        t = lax.axis_index("c") * 16 + lax.axis_index("s")
        rbase = t * rows

        def bucket_fn(v):
            key = _mono_key(v)
            return (key >> 22) + 512, None

        for a, ref in enumerate((l0, l1, l2)):
            pltpu.sync_copy(ref.at[pl.ds(rbase, rows), :], stage)
            _hist_core(stage, hc, hs, mc, ms, rows * 8, bucket_fn)
            pltpu.sync_copy(mc, cnt_out.at[pl.ds(t, 1), pl.ds(a * NB, NB)])
            pltpu.sync_copy(ms, sum_out.at[pl.ds(t, 1), pl.ds(a * NB, NB)])

    return hist1


def _make_hist2(n):
    rows = n // LANES // NTILES
    mesh = plsc.VectorSubcoreMesh(core_axis_name="c", subcore_axis_name="s")

    @functools.partial(
        pl.kernel,
        mesh=mesh,
        compiler_params=pltpu.CompilerParams(needs_layout_passes=False),
        out_type=[
            jax.ShapeDtypeStruct((NTILES, 3 * NB), jnp.int32),
            jax.ShapeDtypeStruct((NTILES, 3 * NB), jnp.float32),
        ],
        scratch_types=[
            pltpu.VMEM((rows, LANES), jnp.float32),
            pltpu.VMEM((8, 128), jnp.float32),
            pltpu.VMEM((NB * NCOPY,), jnp.int32),
            pltpu.VMEM((NB * NCOPY,), jnp.float32),
            pltpu.VMEM((1, NB), jnp.int32),
            pltpu.VMEM((1, NB), jnp.float32),
        ],
    )
    def hist2(l0, l1, l2, st, cnt_out, sum_out, stage, stv, hc, hs, mc, ms):
        t = lax.axis_index("c") * 16 + lax.axis_index("s")
        rbase = t * rows
        pltpu.sync_copy(st, stv)

        for a, ref in enumerate((l0, l1, l2)):
            b1 = stv[a, pl.ds(0, 16)][0].astype(jnp.int32)

            def bucket_fn(v, b1=b1):
                key = _mono_key(v)
                match = ((key >> 22) + 512) == b1
                return (key >> 12) & (NB - 1), match

            pltpu.sync_copy(ref.at[pl.ds(rbase, rows), :], stage)
            _hist_core(stage, hc, hs, mc, ms, rows * 8, bucket_fn)
            pltpu.sync_copy(mc, cnt_out.at[pl.ds(t, 1), pl.ds(a * NB, NB)])
            pltpu.sync_copy(ms, sum_out.at[pl.ds(t, 1), pl.ds(a * NB, NB)])

    return hist2


def _suffix(x, tri):
    return lax.dot_general(x, tri, (((1,), (1,)), ((), ())),
                           preferred_element_type=jnp.float32)


def _tri():
    ii = lax.broadcasted_iota(jnp.int32, (NB, NB), 0)
    jj = lax.broadcasted_iota(jnp.int32, (NB, NB), 1)
    return (jj >= ii).astype(jnp.float32)


def _scan1_body(k0, cnt_ref, sum_ref, st_ref):
    cnt_all = cnt_ref[...].astype(jnp.float32)
    sum_all = sum_ref[...]
    tri = _tri()
    jr = lax.broadcasted_iota(jnp.int32, (1, NB), 1)
    rows = []
    for a in range(3):
        ca = jnp.sum(cnt_all[:, a * NB:(a + 1) * NB], axis=0, keepdims=True)
        sa = jnp.sum(sum_all[:, a * NB:(a + 1) * NB], axis=0, keepdims=True)
        rc = _suffix(ca, tri)
        if a == 0:
            k = jnp.float32(k0)
        else:
            cv = jnp.sum(jnp.where(jr >= 512, ca, 0.0))
            k = jnp.floor(cv * 7.0 / 10.0)
        b1 = jnp.sum((rc >= k).astype(jnp.float32)) - 1.0
        gt = jr.astype(jnp.float32) > b1
        count_above = jnp.sum(jnp.where(gt, ca, 0.0))
        sum_above = jnp.sum(jnp.where(gt, sa, 0.0))
        rows.append((b1, count_above, sum_above, k))
    r8 = lax.broadcasted_iota(jnp.int32, (8, 128), 0)
    c128 = lax.broadcasted_iota(jnp.int32, (8, 128), 1)
    st = jnp.zeros((8, 128), jnp.float32)
    for a in range(3):
        for j, val in enumerate(rows[a]):
            st = jnp.where((r8 == a) & (c128 == j), val, st)
    st_ref[...] = st


def _scan1(cnt1, sum1, k0):
    return pl.pallas_call(
        functools.partial(_scan1_body, k0),
        out_shape=jax.ShapeDtypeStruct((8, 128), jnp.float32),
    )(cnt1, sum1)


def _scan2_body(cnt_ref, sum_ref, st_ref, out_ref):
    cnt_all = cnt_ref[...].astype(jnp.float32)
    sum_all = sum_ref[...]
    st = st_ref[...]
    tri = _tri()
    jr = lax.broadcasted_iota(jnp.int32, (1, NB), 1)
    r8 = lax.broadcasted_iota(jnp.int32, (8, 128), 0)
    c128 = lax.broadcasted_iota(jnp.int32, (8, 128), 1)
    losses = []
    for a in range(3):
        def pick(j):
            return jnp.sum(jnp.where((r8 == a) & (c128 == j), st, 0.0))
        count_above, sum_above, k = pick(1), pick(2), pick(3)
        ca = jnp.sum(cnt_all[:, a * NB:(a + 1) * NB], axis=0, keepdims=True)
        sa = jnp.sum(sum_all[:, a * NB:(a + 1) * NB], axis=0, keepdims=True)
        rc = _suffix(ca, tri)
        b2 = jnp.sum((count_above + rc >= k).astype(jnp.float32)) - 1.0
        jf = jr.astype(jnp.float32)
        count_gt = count_above + jnp.sum(jnp.where(jf > b2, ca, 0.0))
        sum_gt = sum_above + jnp.sum(jnp.where(jf > b2, sa, 0.0))
        count_eq = jnp.sum(jnp.where(jf == b2, ca, 0.0))
        sum_eq = jnp.sum(jnp.where(jf == b2, sa, 0.0))
        r = k - count_gt
        losses.append((sum_gt + r * (sum_eq / count_eq)) / k)
    total = losses[0] + 0.5 * losses[1] + 0.5 * losses[2]
    vals = (total, losses[0], losses[1], losses[2])
    out = jnp.zeros((8, 128), jnp.float32)
    for j, v in enumerate(vals):
        out = jnp.where((r8 == 0) & (c128 == j), v, out)
    out_ref[...] = out


def _scan2(cnt2, sum2, st):
    return pl.pallas_call(
        _scan2_body,
        out_shape=jax.ShapeDtypeStruct((8, 128), jnp.float32),
    )(cnt2, sum2, st)


def kernel(pred, gt):
    n = pred.shape[0]
    k0 = int(0.7 * n)
    l_cls, l_bb, l_lm = _losses(pred, gt)
    cnt1, sum1 = _make_hist1(n)(l_cls, l_bb, l_lm)
    st = _scan1(cnt1, sum1, k0)
    cnt2, sum2 = _make_hist2(n)(l_cls, l_bb, l_lm, st)
    out = _scan2(cnt2, sum2, st)
    return out[0, 0], out[0, 1], out[0, 2], out[0, 3]


# speedup vs baseline: 2.7665x; 1.0832x over previous
import functools

import jax
import jax.numpy as jnp
from jax import lax
from jax.experimental import pallas as pl
from jax.experimental.pallas import tpu as pltpu
from jax.experimental.pallas import tpu_sc as plsc

NB = 1024
NCOPY = 16
BLK = 4096
NTILES = 32
LANES = 128


BS = 4096


def _loss_body(pred_ref, gt_ref, wp_ref, wsq_ref, cls_ref, bb_ref, lm_ref):
    j = pl.program_id(1)
    p = pred_ref[...]
    g = gt_ref[...]
    d = p - g
    sq = d * d
    tdot = lambda w, m: lax.dot_general(w, m, (((1,), (1,)), ((), ())),
                                        preferred_element_type=jnp.float32)
    xq = tdot(wp_ref[...], p)
    gq = tdot(wp_ref[...], g)
    mq = tdot(wsq_ref[...], sq)
    x = xq[0:1, :]
    lab = gq[0:1, :]
    lb = mq[0:1, :]
    ll = mq[1:2, :]
    y = jnp.where(lab == 1.0, 1.0, 0.0)
    bce = jnp.maximum(x, 0.0) - x * y + jnp.log1p(jnp.exp(-jnp.abs(x)))
    bbm = jnp.where(jnp.logical_or(lab == 1.0, lab == -1.0), lb, -1.0)
    lmm = jnp.where(lab == -2.0, ll, -1.0)
    for h in range(2):
        row = pl.ds(j * 2 + h, 1)
        sl = (slice(None), slice(h * BS, (h + 1) * BS))
        cls_ref[row, :] = bce[sl]
        bb_ref[row, :] = bbm[sl]
        lm_ref[row, :] = lmm[sl]


def _losses(pred, gt):
    n = pred.shape[0]
    cc = jnp.arange(15)
    row = jnp.arange(8)
    wp = ((row[:, None] == 0) & (cc == 0)[None, :]).astype(jnp.float32)
    wsq = ((row[:, None] == 0) * ((cc >= 1) & (cc <= 4))[None, :] * 0.25
           + (row[:, None] == 1) * (cc >= 5)[None, :] * 0.1
           ).astype(jnp.float32)
    out = jax.ShapeDtypeStruct((n // BS, BS), jnp.float32)
    cidx = lambda i, j: (0, 0)
    return pl.pallas_call(
        _loss_body,
        grid=(n // BS // 8, 4),
        in_specs=[
            pl.BlockSpec((2 * BS, 15), lambda i, j: (i * 4 + j, 0)),
            pl.BlockSpec((2 * BS, 15), lambda i, j: (i * 4 + j, 0)),
            pl.BlockSpec((8, 15), cidx),
            pl.BlockSpec((8, 15), cidx),
        ],
        out_specs=[pl.BlockSpec((8, BS), lambda i, j: (i, 0))] * 3,
        out_shape=[out, out, out],
    )(pred, gt, wp, wsq)


def _mono_key(v):
    bits = lax.bitcast_convert_type(v, jnp.int32)
    return jnp.where(bits < 0, bits ^ jnp.int32(0x7FFFFFFF), bits)


def _hist_core(stage, hc, hs, mc, ms, n_vec, bucket_fn):
    z16i = jnp.zeros((16,), jnp.int32)
    z16f = jnp.zeros((16,), jnp.float32)
    ones = jnp.ones((16,), jnp.int32)
    lane_off = lax.iota(jnp.int32, 16) * NB

    def zero_body(i, _):
        hc[pl.ds(i * 16, 16)] = z16i
        hs[pl.ds(i * 16, 16)] = z16f
        return 0
    lax.fori_loop(0, NB * NCOPY // 16, zero_body, 0)

    def scat_body(i, _):
        v = stage[i >> 8, pl.ds((i & 255) * 16, 16)]
        b, mask = bucket_fn(v)
        idx = lane_off + b
        plsc.addupdate_scatter(hc, [idx], ones, mask=mask)
        plsc.addupdate_scatter(hs, [idx], v, mask=mask)
        return 0
    lax.fori_loop(0, n_vec, scat_body, 0)

    def merge_body(i, _):
        acc_c = hc[pl.ds(i * 16, 16)]
        acc_s = hs[pl.ds(i * 16, 16)]
        for l in range(1, NCOPY):
            acc_c = acc_c + hc[pl.ds(l * NB + i * 16, 16)]
            acc_s = acc_s + hs[pl.ds(l * NB + i * 16, 16)]
        mc[0, pl.ds(i * 16, 16)] = acc_c
        ms[0, pl.ds(i * 16, 16)] = acc_s
        return 0
    lax.fori_loop(0, NB // 16, merge_body, 0)


def _make_hist1(n):
    rows = n // BS // NTILES
    mesh = plsc.VectorSubcoreMesh(core_axis_name="c", subcore_axis_name="s")

    @functools.partial(
        pl.kernel,
        mesh=mesh,
        compiler_params=pltpu.CompilerParams(needs_layout_passes=False),
        out_type=[
            jax.ShapeDtypeStruct((NTILES, 3 * NB), jnp.int32),
            jax.ShapeDtypeStruct((NTILES, 3 * NB), jnp.float32),
        ],
        scratch_types=[
            pltpu.VMEM((rows, BS), jnp.float32),
            pltpu.VMEM((NB * NCOPY,), jnp.int32),
            pltpu.VMEM((NB * NCOPY,), jnp.float32),
            pltpu.VMEM((1, NB), jnp.int32),
            pltpu.VMEM((1, NB), jnp.float32),
        ],
    )
    def hist1(l0, l1, l2, cnt_out, sum_out, stage, hc, hs, mc, ms):
        t = lax.axis_index("c") * 16 + lax.axis_index("s")
        rbase = t * rows

        def bucket_fn(v):
            key = _mono_key(v)
            return (key >> 22) + 512, None

        for a, ref in enumerate((l0, l1, l2)):
            pltpu.sync_copy(ref.at[pl.ds(rbase, rows), :], stage)
            _hist_core(stage, hc, hs, mc, ms, rows * BS // 16, bucket_fn)
            pltpu.sync_copy(mc, cnt_out.at[pl.ds(t, 1), pl.ds(a * NB, NB)])
            pltpu.sync_copy(ms, sum_out.at[pl.ds(t, 1), pl.ds(a * NB, NB)])

    return hist1


def _make_hist2(n):
    rows = n // BS // NTILES
    mesh = plsc.VectorSubcoreMesh(core_axis_name="c", subcore_axis_name="s")

    @functools.partial(
        pl.kernel,
        mesh=mesh,
        compiler_params=pltpu.CompilerParams(needs_layout_passes=False),
        out_type=[
            jax.ShapeDtypeStruct((NTILES, 3 * NB), jnp.int32),
            jax.ShapeDtypeStruct((NTILES, 3 * NB), jnp.float32),
        ],
        scratch_types=[
            pltpu.VMEM((rows, BS), jnp.float32),
            pltpu.VMEM((8, 128), jnp.float32),
            pltpu.VMEM((NB * NCOPY,), jnp.int32),
            pltpu.VMEM((NB * NCOPY,), jnp.float32),
            pltpu.VMEM((1, NB), jnp.int32),
            pltpu.VMEM((1, NB), jnp.float32),
        ],
    )
    def hist2(l0, l1, l2, st, cnt_out, sum_out, stage, stv, hc, hs, mc, ms):
        t = lax.axis_index("c") * 16 + lax.axis_index("s")
        rbase = t * rows
        pltpu.sync_copy(st, stv)

        for a, ref in enumerate((l0, l1, l2)):
            b1 = stv[a, pl.ds(0, 16)][0].astype(jnp.int32)

            def bucket_fn(v, b1=b1):
                key = _mono_key(v)
                match = ((key >> 22) + 512) == b1
                return (key >> 12) & (NB - 1), match

            pltpu.sync_copy(ref.at[pl.ds(rbase, rows), :], stage)
            _hist_core(stage, hc, hs, mc, ms, rows * BS // 16, bucket_fn)
            pltpu.sync_copy(mc, cnt_out.at[pl.ds(t, 1), pl.ds(a * NB, NB)])
            pltpu.sync_copy(ms, sum_out.at[pl.ds(t, 1), pl.ds(a * NB, NB)])

    return hist2


def _suffix(x, tri):
    return lax.dot_general(x, tri, (((1,), (1,)), ((), ())),
                           preferred_element_type=jnp.float32)


def _tri():
    ii = lax.broadcasted_iota(jnp.int32, (NB, NB), 0)
    jj = lax.broadcasted_iota(jnp.int32, (NB, NB), 1)
    return (jj >= ii).astype(jnp.float32)


def _scan1_body(k0, cnt_ref, sum_ref, st_ref):
    cnt_all = cnt_ref[...].astype(jnp.float32)
    sum_all = sum_ref[...]
    tri = _tri()
    jr = lax.broadcasted_iota(jnp.int32, (1, NB), 1)
    rows = []
    for a in range(3):
        ca = jnp.sum(cnt_all[:, a * NB:(a + 1) * NB], axis=0, keepdims=True)
        sa = jnp.sum(sum_all[:, a * NB:(a + 1) * NB], axis=0, keepdims=True)
        rc = _suffix(ca, tri)
        if a == 0:
            k = jnp.float32(k0)
        else:
            cv = jnp.sum(jnp.where(jr >= 512, ca, 0.0))
            k = jnp.floor(cv * 7.0 / 10.0)
        b1 = jnp.sum((rc >= k).astype(jnp.float32)) - 1.0
        gt = jr.astype(jnp.float32) > b1
        count_above = jnp.sum(jnp.where(gt, ca, 0.0))
        sum_above = jnp.sum(jnp.where(gt, sa, 0.0))
        rows.append((b1, count_above, sum_above, k))
    r8 = lax.broadcasted_iota(jnp.int32, (8, 128), 0)
    c128 = lax.broadcasted_iota(jnp.int32, (8, 128), 1)
    st = jnp.zeros((8, 128), jnp.float32)
    for a in range(3):
        for j, val in enumerate(rows[a]):
            st = jnp.where((r8 == a) & (c128 == j), val, st)
    st_ref[...] = st


def _scan1(cnt1, sum1, k0):
    return pl.pallas_call(
        functools.partial(_scan1_body, k0),
        out_shape=jax.ShapeDtypeStruct((8, 128), jnp.float32),
    )(cnt1, sum1)


def _scan2_body(cnt_ref, sum_ref, st_ref, out_ref):
    cnt_all = cnt_ref[...].astype(jnp.float32)
    sum_all = sum_ref[...]
    st = st_ref[...]
    tri = _tri()
    jr = lax.broadcasted_iota(jnp.int32, (1, NB), 1)
    r8 = lax.broadcasted_iota(jnp.int32, (8, 128), 0)
    c128 = lax.broadcasted_iota(jnp.int32, (8, 128), 1)
    losses = []
    for a in range(3):
        def pick(j):
            return jnp.sum(jnp.where((r8 == a) & (c128 == j), st, 0.0))
        count_above, sum_above, k = pick(1), pick(2), pick(3)
        ca = jnp.sum(cnt_all[:, a * NB:(a + 1) * NB], axis=0, keepdims=True)
        sa = jnp.sum(sum_all[:, a * NB:(a + 1) * NB], axis=0, keepdims=True)
        rc = _suffix(ca, tri)
        b2 = jnp.sum((count_above + rc >= k).astype(jnp.float32)) - 1.0
        jf = jr.astype(jnp.float32)
        count_gt = count_above + jnp.sum(jnp.where(jf > b2, ca, 0.0))
        sum_gt = sum_above + jnp.sum(jnp.where(jf > b2, sa, 0.0))
        count_eq = jnp.sum(jnp.where(jf == b2, ca, 0.0))
        sum_eq = jnp.sum(jnp.where(jf == b2, sa, 0.0))
        r = k - count_gt
        losses.append((sum_gt + r * (sum_eq / count_eq)) / k)
    total = losses[0] + 0.5 * losses[1] + 0.5 * losses[2]
    vals = (total, losses[0], losses[1], losses[2])
    out = jnp.zeros((8, 128), jnp.float32)
    for j, v in enumerate(vals):
        out = jnp.where((r8 == 0) & (c128 == j), v, out)
    out_ref[...] = out


def _scan2(cnt2, sum2, st):
    return pl.pallas_call(
        _scan2_body,
        out_shape=jax.ShapeDtypeStruct((8, 128), jnp.float32),
    )(cnt2, sum2, st)


def kernel(pred, gt):
    n = pred.shape[0]
    k0 = int(0.7 * n)
    l_cls, l_bb, l_lm = _losses(pred, gt)
    cnt1, sum1 = _make_hist1(n)(l_cls, l_bb, l_lm)
    st = _scan1(cnt1, sum1, k0)
    cnt2, sum2 = _make_hist2(n)(l_cls, l_bb, l_lm, st)
    out = _scan2(cnt2, sum2, st)
    return out[0, 0], out[0, 1], out[0, 2], out[0, 3]
